# Initial kernel scaffold; baseline (speedup 1.0000x reference)
import jax
import jax.numpy as jnp
from jax.experimental import pallas as pl


def kernel(y, x, f_y, W1, b1, W2, b2):
    raise NotImplementedError("write your pallas kernel here")


# speedup vs baseline: 2.1866x; 2.1866x over previous
import functools

import jax
import jax.numpy as jnp
from jax.experimental import pallas as pl
from jax.experimental.pallas import tpu as pltpu

_R = 0.072
_R2 = _R * _R
_BM = 16
_BN = 128
_GRID = 8


def _ab_body(ymp_ref, xmp_ref, w1y_ref, w1x_ref, b1_ref, a_ref, b_ref):
    a_ref[:, :] = (
        jnp.dot(ymp_ref[:, :], w1y_ref[:, :],
                preferred_element_type=jnp.float32,
                precision=jax.lax.Precision.HIGHEST)
        + b1_ref[0:1, :]
    )
    b_ref[:, :] = jnp.dot(xmp_ref[:, :], w1x_ref[:, :],
                          preferred_element_type=jnp.float32,
                          precision=jax.lax.Precision.HIGHEST)


def _main_body(nj, xs8_ref, bs_ref, ys8_ref, as_ref, fs_ref, w2_ref, b2_ref,
               out_ref, acc_ref, fsum_ref, cnt_ref, mbuf_ref):
    acc_ref[:, :] = jnp.zeros_like(acc_ref)
    fsum_ref[:, :] = jnp.zeros_like(fsum_ref)
    cnt_ref[:, :] = jnp.zeros_like(cnt_ref)

    x8 = xs8_ref[:, :]
    bq = bs_ref[:, :]
    w2 = w2_ref[:, :]

    def jstep(j, carry):
        yt = ys8_ref[pl.ds(j * _BN, _BN), :]
        d2 = jax.lax.dot_general(
            x8, yt, (((1,), (1,)), ((), ())),
            preferred_element_type=jnp.float32,
            precision=jax.lax.Precision.HIGHEST)
        mask = d2 <= _R2

        @pl.when(jnp.any(mask))
        def _():
            at = as_ref[pl.ds(j * _BN, _BN), :]
            ft = fs_ref[pl.ds(j * _BN, _BN), :]
            mf = mask.astype(jnp.float32)
            mbuf_ref[:, :] = mf

            def qstep(q, c):
                row = bs_ref[pl.ds(q, 1), :]
                h = jax.nn.gelu(at + row)
                k = jnp.dot(h, w2, preferred_element_type=jnp.float32)
                kf = k * ft
                mrow = mbuf_ref[pl.ds(q, 1), :]
                contrib = jnp.dot(mrow, kf,
                                  preferred_element_type=jnp.float32)
                acc_ref[pl.ds(q, 1), :] += contrib
                return c

            jax.lax.fori_loop(0, _BM, qstep, 0)
            fsum_ref[:, :] += jnp.dot(mf, ft,
                                      preferred_element_type=jnp.float32)
            cnt_ref[:, :] += jnp.sum(mf, axis=1, keepdims=True)

        return carry

    jax.lax.fori_loop(0, nj, jstep, 0)

    cntc = jnp.maximum(cnt_ref[:, 0:1], 1.0)
    out_ref[:, :] = (acc_ref[:, :] + b2_ref[0:1, :] * fsum_ref[:, :]) / cntc


def kernel(y, x, f_y, W1, b1, W2, b2):
    m = x.shape[0]
    n = y.shape[0]
    c = f_y.shape[1]

    def cell_id(p):
        ci = jnp.clip((p * _GRID).astype(jnp.int32), 0, _GRID - 1)
        return (ci[:, 0] * _GRID + ci[:, 1]) * _GRID + ci[:, 2]

    perm_x = jnp.argsort(cell_id(x))
    perm_y = jnp.argsort(cell_id(y))
    xs = x[perm_x]
    ys = y[perm_y]
    fs = f_y[perm_y]
    inv_x = jnp.argsort(perm_x)

    pad = ((max(m, n) + 127) // 128) * 128
    m_pad = pad
    n_pad = pad
    big = jnp.float32(1e9)

    xn2 = jnp.sum(xs * xs, axis=1)
    yn2 = jnp.sum(ys * ys, axis=1)
    xs8 = jnp.zeros((m_pad, 8), jnp.float32)
    xs8 = xs8.at[:m, 0:3].set(-2.0 * xs)
    xs8 = xs8.at[:m, 3].set(xn2)
    xs8 = xs8.at[:m, 4].set(1.0)
    xs8 = xs8.at[m:, 3].set(big)
    xs8 = xs8.at[m:, 4].set(1.0)
    ys8 = jnp.zeros((n_pad, 8), jnp.float32)
    ys8 = ys8.at[:n, 0:3].set(ys)
    ys8 = ys8.at[:n, 3].set(1.0)
    ys8 = ys8.at[:n, 4].set(yn2)
    ys8 = ys8.at[n:, 3].set(1.0)
    ys8 = ys8.at[n:, 4].set(big)

    ymp = jnp.zeros((n_pad, 8), jnp.float32).at[:n, 0:3].set(ys)
    xmp = jnp.zeros((m_pad, 8), jnp.float32).at[:m, 0:3].set(xs)
    w1y = jnp.zeros((8, 512), jnp.float32).at[0:3, :].set(W1[0:3])
    w1x = jnp.zeros((8, 512), jnp.float32).at[0:3, :].set(W1[3:6])
    b1r = jnp.broadcast_to(b1[None, :], (8, 512))
    b2r = jnp.broadcast_to(b2[None, :], (8, c))

    assert m_pad == n_pad
    tb = 256
    a_s, b_s = pl.pallas_call(
        _ab_body,
        grid=(n_pad // tb,),
        in_specs=[
            pl.BlockSpec((tb, 8), lambda i: (i, 0)),
            pl.BlockSpec((tb, 8), lambda i: (i, 0)),
            pl.BlockSpec((8, 512), lambda i: (0, 0)),
            pl.BlockSpec((8, 512), lambda i: (0, 0)),
            pl.BlockSpec((8, 512), lambda i: (0, 0)),
        ],
        out_specs=[
            pl.BlockSpec((tb, 512), lambda i: (i, 0)),
            pl.BlockSpec((tb, 512), lambda i: (i, 0)),
        ],
        out_shape=[
            jax.ShapeDtypeStruct((n_pad, 512), jnp.float32),
            jax.ShapeDtypeStruct((m_pad, 512), jnp.float32),
        ],
    )(ymp, xmp, w1y, w1x, b1r)

    fsp = jnp.zeros((n_pad, c), jnp.float32).at[:n].set(fs)

    nj = n_pad // _BN
    out_s = pl.pallas_call(
        functools.partial(_main_body, nj),
        grid=(m_pad // _BM,),
        in_specs=[
            pl.BlockSpec((_BM, 8), lambda i: (i, 0)),
            pl.BlockSpec((_BM, 512), lambda i: (i, 0)),
            pl.BlockSpec((n_pad, 8), lambda i: (0, 0)),
            pl.BlockSpec((n_pad, 512), lambda i: (0, 0)),
            pl.BlockSpec((n_pad, c), lambda i: (0, 0)),
            pl.BlockSpec((512, 256), lambda i: (0, 0)),
            pl.BlockSpec((8, c), lambda i: (0, 0)),
        ],
        out_specs=pl.BlockSpec((_BM, c), lambda i: (i, 0)),
        out_shape=jax.ShapeDtypeStruct((m_pad, c), jnp.float32),
        scratch_shapes=[
            pltpu.VMEM((_BM, c), jnp.float32),
            pltpu.VMEM((_BM, c), jnp.float32),
            pltpu.VMEM((_BM, 128), jnp.float32),
            pltpu.VMEM((_BM, _BN), jnp.float32),
        ],
    )(xs8, b_s, ys8, a_s, fsp, W2, b2r)

    return out_s[:m][inv_x]


# speedup vs baseline: 3.9336x; 1.7990x over previous
import functools

import jax
import jax.numpy as jnp
from jax.experimental import pallas as pl
from jax.experimental.pallas import tpu as pltpu

_R = 0.072
_R2 = _R * _R
_BM = 16
_BN = 128
_GRID = 8


def _ab_body(ymp_ref, xmp_ref, w1y_ref, w1x_ref, b1_ref, a_ref, b_ref):
    a_ref[:, :] = (
        jnp.dot(ymp_ref[:, :], w1y_ref[:, :],
                preferred_element_type=jnp.float32,
                precision=jax.lax.Precision.HIGHEST)
        + b1_ref[0:1, :]
    )
    b_ref[:, :] = jnp.dot(xmp_ref[:, :], w1x_ref[:, :],
                          preferred_element_type=jnp.float32,
                          precision=jax.lax.Precision.HIGHEST)


def _main_body(nj, xs8_ref, bs_ref, ys8_ref, as_ref, fs_ref, w2_ref, b2_ref,
               out_ref, acc_ref, fsum_ref, cnt_ref):
    acc_ref[:, :] = jnp.zeros_like(acc_ref)
    fsum_ref[:, :] = jnp.zeros_like(fsum_ref)
    cnt_ref[:, :] = jnp.zeros_like(cnt_ref)

    x8 = xs8_ref[:, :]
    bq = bs_ref[:, :]
    w2 = w2_ref[:, :]

    def jstep(j, carry):
        yt = ys8_ref[pl.ds(j * _BN, _BN), :]
        d2 = jax.lax.dot_general(
            x8, yt, (((1,), (1,)), ((), ())),
            preferred_element_type=jnp.float32,
            precision=jax.lax.Precision.HIGHEST)
        mask = d2 <= _R2

        @pl.when(jnp.any(mask))
        def _():
            at = as_ref[pl.ds(j * _BN, _BN), :]
            ft = fs_ref[pl.ds(j * _BN, _BN), :]
            mf = mask.astype(jnp.float32)

            h3 = jax.nn.gelu(bq[:, None, :] + at[None, :, :])
            h2 = h3.reshape(_BM * _BN, 512)
            kk = jnp.dot(h2, w2, preferred_element_type=jnp.float32)
            k3 = kk.reshape(_BM, _BN, 256)
            k3 = k3 * ft[None, :, :] * mf[:, :, None]
            acc_ref[:, :] += jnp.sum(k3, axis=1)
            fsum_ref[:, :] += jnp.dot(mf, ft,
                                      preferred_element_type=jnp.float32)
            cnt_ref[:, :] += jnp.sum(mf, axis=1, keepdims=True)

        return carry

    jax.lax.fori_loop(0, nj, jstep, 0)

    cntc = jnp.maximum(cnt_ref[:, 0:1], 1.0)
    out_ref[:, :] = (acc_ref[:, :] + b2_ref[0:1, :] * fsum_ref[:, :]) / cntc


def kernel(y, x, f_y, W1, b1, W2, b2):
    m = x.shape[0]
    n = y.shape[0]
    c = f_y.shape[1]

    def cell_id(p):
        ci = jnp.clip((p * _GRID).astype(jnp.int32), 0, _GRID - 1)
        return (ci[:, 0] * _GRID + ci[:, 1]) * _GRID + ci[:, 2]

    perm_x = jnp.argsort(cell_id(x))
    perm_y = jnp.argsort(cell_id(y))
    xs = x[perm_x]
    ys = y[perm_y]
    fs = f_y[perm_y]
    inv_x = jnp.argsort(perm_x)

    pad = ((max(m, n) + 127) // 128) * 128
    m_pad = pad
    n_pad = pad
    big = jnp.float32(1e9)

    xn2 = jnp.sum(xs * xs, axis=1)
    yn2 = jnp.sum(ys * ys, axis=1)
    xs8 = jnp.zeros((m_pad, 8), jnp.float32)
    xs8 = xs8.at[:m, 0:3].set(-2.0 * xs)
    xs8 = xs8.at[:m, 3].set(xn2)
    xs8 = xs8.at[:m, 4].set(1.0)
    xs8 = xs8.at[m:, 3].set(big)
    xs8 = xs8.at[m:, 4].set(1.0)
    ys8 = jnp.zeros((n_pad, 8), jnp.float32)
    ys8 = ys8.at[:n, 0:3].set(ys)
    ys8 = ys8.at[:n, 3].set(1.0)
    ys8 = ys8.at[:n, 4].set(yn2)
    ys8 = ys8.at[n:, 3].set(1.0)
    ys8 = ys8.at[n:, 4].set(big)

    ymp = jnp.zeros((n_pad, 8), jnp.float32).at[:n, 0:3].set(ys)
    xmp = jnp.zeros((m_pad, 8), jnp.float32).at[:m, 0:3].set(xs)
    w1y = jnp.zeros((8, 512), jnp.float32).at[0:3, :].set(W1[0:3])
    w1x = jnp.zeros((8, 512), jnp.float32).at[0:3, :].set(W1[3:6])
    b1r = jnp.broadcast_to(b1[None, :], (8, 512))
    b2r = jnp.broadcast_to(b2[None, :], (8, c))

    assert m_pad == n_pad
    tb = 256
    a_s, b_s = pl.pallas_call(
        _ab_body,
        grid=(n_pad // tb,),
        in_specs=[
            pl.BlockSpec((tb, 8), lambda i: (i, 0)),
            pl.BlockSpec((tb, 8), lambda i: (i, 0)),
            pl.BlockSpec((8, 512), lambda i: (0, 0)),
            pl.BlockSpec((8, 512), lambda i: (0, 0)),
            pl.BlockSpec((8, 512), lambda i: (0, 0)),
        ],
        out_specs=[
            pl.BlockSpec((tb, 512), lambda i: (i, 0)),
            pl.BlockSpec((tb, 512), lambda i: (i, 0)),
        ],
        out_shape=[
            jax.ShapeDtypeStruct((n_pad, 512), jnp.float32),
            jax.ShapeDtypeStruct((m_pad, 512), jnp.float32),
        ],
    )(ymp, xmp, w1y, w1x, b1r)

    fsp = jnp.zeros((n_pad, c), jnp.float32).at[:n].set(fs)

    nj = n_pad // _BN
    out_s = pl.pallas_call(
        functools.partial(_main_body, nj),
        grid=(m_pad // _BM,),
        in_specs=[
            pl.BlockSpec((_BM, 8), lambda i: (i, 0)),
            pl.BlockSpec((_BM, 512), lambda i: (i, 0)),
            pl.BlockSpec((n_pad, 8), lambda i: (0, 0)),
            pl.BlockSpec((n_pad, 512), lambda i: (0, 0)),
            pl.BlockSpec((n_pad, c), lambda i: (0, 0)),
            pl.BlockSpec((512, 256), lambda i: (0, 0)),
            pl.BlockSpec((8, c), lambda i: (0, 0)),
        ],
        out_specs=pl.BlockSpec((_BM, c), lambda i: (i, 0)),
        out_shape=jax.ShapeDtypeStruct((m_pad, c), jnp.float32),
        scratch_shapes=[
            pltpu.VMEM((_BM, c), jnp.float32),
            pltpu.VMEM((_BM, c), jnp.float32),
            pltpu.VMEM((_BM, 128), jnp.float32),
        ],
    )(xs8, b_s, ys8, a_s, fsp, W2, b2r)

    return out_s[:m][inv_x]
